# Initial kernel scaffold; baseline (speedup 1.0000x reference)
import jax
import jax.numpy as jnp
from jax.experimental import pallas as pl


def kernel(x, edge_index, edge_attr, params):
    raise NotImplementedError("write your pallas kernel here")


# speedup vs baseline: 1.5751x; 1.5751x over previous
import functools

import jax
import jax.numpy as jnp
from jax import lax
from jax.experimental import pallas as pl
from jax.experimental.pallas import tpu as pltpu
from jax.experimental.pallas import tpu_sc as plsc

_N = 50000
_E = 800000
_H = 64
_ROW = 128
_NROWS = _E // _ROW
_NROWS_PAD = 6400
_ZCH = 125
_NCH = _N // _ZCH
_BE = 4000
_BN = 2000


def _mesh():
    return plsc.VectorSubcoreMesh(core_axis_name="core", subcore_axis_name="subcore")


def _sc_gather(table, idx2d):
    rows, width = idx2d.shape
    d = table.shape[1]

    @functools.partial(
        pl.kernel,
        out_type=jax.ShapeDtypeStruct((rows * width, d), table.dtype),
        mesh=_mesh(),
    )
    def k(tab_hbm, i_hbm, o_hbm):
        def body(i_vmem, o_vmem):
            pltpu.sync_copy(tab_hbm.at[i_vmem.at[0]], o_vmem)

        pltpu.emit_pipeline(
            body,
            grid=(rows,),
            in_specs=[pl.BlockSpec((1, width), lambda i: (i, 0))],
            out_specs=[pl.BlockSpec((width, d), lambda i: (i, 0))],
            core_axis_name=("core", "subcore"),
            dimension_semantics=(pltpu.PARALLEL,),
        )(i_hbm, o_hbm)

    return k(table, idx2d)


def _sc_scatter_add(msg128, dst2d):
    @functools.partial(
        pl.kernel,
        out_type=jax.ShapeDtypeStruct((4, _N, 128), jnp.float32),
        mesh=_mesh(),
        compiler_params=pltpu.CompilerParams(use_tc_tiling_on_sc=False),
        scratch_types=[
            pltpu.VMEM_SHARED((_N, 16), jnp.float32),
            pltpu.VMEM((1, _ROW), jnp.int32),
            pltpu.VMEM((_ROW, 128), jnp.float32),
            pltpu.VMEM((_ROW, 16), jnp.float32),
            pltpu.VMEM((_ZCH, 16), jnp.float32),
            pltpu.VMEM((_ZCH, 128), jnp.float32),
        ],
    )
    def k(m_hbm, i_hbm, o_hbm, acc, idx_v, m_big, m_v, z_v, stage_v):
        c = lax.axis_index("core")
        s = lax.axis_index("subcore")

        @pl.loop(0, _ZCH)
        def _z(i):
            z_v[i, :] = jnp.zeros((16,), jnp.float32)

        for q_off in range(2):
            q = 2 * c + q_off

            @pl.loop(s, _NCH, step=16)
            def _zero(ch):
                pltpu.sync_copy(z_v, acc.at[pl.ds(ch * _ZCH, _ZCH), :])

            plsc.subcore_barrier()

            @pl.loop(s, _NROWS, step=16)
            def _scat(r):
                pltpu.sync_copy(i_hbm.at[pl.ds(r, 1), :], idx_v)
                pltpu.sync_copy(m_hbm.at[pl.ds(r * _ROW, _ROW), :], m_big)
                for q_s in range(4):
                    @pl.when(q == q_s)
                    def _cp():
                        @pl.loop(0, _ROW)
                        def _row(i):
                            m_v[i, :] = m_big[i, pl.ds(16 * q_s, 16)]
                pltpu.sync_copy(m_v, acc.at[idx_v.at[0]], add=True)

            plsc.subcore_barrier()

            @pl.loop(s, _NCH, step=16)
            def _out(ch):
                pltpu.sync_copy(acc.at[pl.ds(ch * _ZCH, _ZCH), :],
                                stage_v.at[:, pl.ds(0, 16)])
                pltpu.sync_copy(stage_v, o_hbm.at[q, pl.ds(ch * _ZCH, _ZCH), :])

            plsc.subcore_barrier()

    return k(msg128, dst2d)


def _dot(a, b):
    return jnp.dot(a, b, preferred_element_type=jnp.float32)


def _full(a):
    return pl.BlockSpec(a.shape, lambda i: tuple(0 for _ in a.shape))


def _mlp3(xp, w1, b1, w2, b2, w3, b3, blk, pad_left=False):
    r, k0 = xp.shape
    ko = w3.shape[1]

    def body(x_ref, w1r, b1r, w2r, b2r, w3r, b3r, o_ref):
        h = jnp.maximum(_dot(x_ref[...], w1r[...]) + b1r[...], 0.0)
        h = jnp.maximum(_dot(h, w2r[...]) + b2r[...], 0.0)
        o = _dot(h, w3r[...]) + b3r[...]
        if pad_left:
            o = jnp.concatenate([jnp.zeros_like(o), o], axis=1)
        o_ref[...] = o

    if pad_left:
        ko *= 2
    return pl.pallas_call(
        body,
        grid=(r // blk,),
        in_specs=[pl.BlockSpec((blk, k0), lambda i: (i, 0)),
                  _full(w1), _full(b1), _full(w2), _full(b2), _full(w3), _full(b3)],
        out_specs=pl.BlockSpec((blk, ko), lambda i: (i, 0)),
        out_shape=jax.ShapeDtypeStruct((r, ko), jnp.float32),
    )(xp, w1, b1, w2, b2, w3, b3)


def _tc_node_in(xp, w1, b1, w2, b2, w3, b3, w1a):
    r, k0 = xp.shape

    def body(x_ref, w1r, b1r, w2r, b2r, w3r, b3r, w1ar, nf_ref, p_ref):
        h = jnp.maximum(_dot(x_ref[...], w1r[...]) + b1r[...], 0.0)
        h = jnp.maximum(_dot(h, w2r[...]) + b2r[...], 0.0)
        nf = _dot(h, w3r[...]) + b3r[...]
        nf_ref[...] = nf
        p = _dot(nf, w1ar[...])
        p_ref[...] = jnp.concatenate([p, jnp.zeros_like(p)], axis=1)

    return pl.pallas_call(
        body,
        grid=(r // _BN,),
        in_specs=[pl.BlockSpec((_BN, k0), lambda i: (i, 0)),
                  _full(w1), _full(b1), _full(w2), _full(b2), _full(w3),
                  _full(b3), _full(w1a)],
        out_specs=[pl.BlockSpec((_BN, _H), lambda i: (i, 0)),
                   pl.BlockSpec((_BN, 2 * _H), lambda i: (i, 0))],
        out_shape=[jax.ShapeDtypeStruct((r, _H), jnp.float32),
                   jax.ShapeDtypeStruct((r, 2 * _H), jnp.float32)],
    )(xp, w1, b1, w2, b2, w3, b3, w1a)


def _tc_edge(g128, me, w1b, b1, w2, b2, w3, b3):
    def body(g_ref, me_ref, w1br, b1r, w2r, b2r, w3r, b3r, o_ref):
        p = g_ref[...][:, :_H]
        ef = me_ref[...][:, _H:]
        h = jnp.maximum(p + _dot(ef, w1br[...]) + b1r[...], 0.0)
        h = jnp.maximum(_dot(h, w2r[...]) + b2r[...], 0.0)
        m = _dot(h, w3r[...]) + b3r[...]
        o_ref[...] = jnp.concatenate([m, ef + m], axis=1)

    return pl.pallas_call(
        body,
        grid=(_E // _BE,),
        in_specs=[pl.BlockSpec((_BE, 2 * _H), lambda i: (i, 0)),
                  pl.BlockSpec((_BE, 2 * _H), lambda i: (i, 0)),
                  _full(w1b), _full(b1), _full(w2), _full(b2), _full(w3), _full(b3)],
        out_specs=pl.BlockSpec((_BE, 2 * _H), lambda i: (i, 0)),
        out_shape=jax.ShapeDtypeStruct((_E, 2 * _H), jnp.float32),
    )(g128, me, w1b, b1, w2, b2, w3, b3)


def _tc_node(nf, ag4, w1, b1, w2, b2, w3, b3, w1a):
    def body(nf_ref, ag_ref, w1r, b1r, w2r, b2r, w3r, b3r, w1ar, nfn_ref, p_ref):
        ag = ag_ref[...]
        cat = jnp.concatenate(
            [nf_ref[...], ag[0, :, :16], ag[1, :, :16], ag[2, :, :16],
             ag[3, :, :16]], axis=1)
        h = jnp.maximum(_dot(cat, w1r[...]) + b1r[...], 0.0)
        h = jnp.maximum(_dot(h, w2r[...]) + b2r[...], 0.0)
        nfn = nf_ref[...] + _dot(h, w3r[...]) + b3r[...]
        nfn_ref[...] = nfn
        p = _dot(nfn, w1ar[...])
        p_ref[...] = jnp.concatenate([p, jnp.zeros_like(p)], axis=1)

    return pl.pallas_call(
        body,
        grid=(_N // _BN,),
        in_specs=[pl.BlockSpec((_BN, _H), lambda i: (i, 0)),
                  pl.BlockSpec((4, _BN, 128), lambda i: (0, i, 0)),
                  _full(w1), _full(b1), _full(w2), _full(b2), _full(w3),
                  _full(b3), _full(w1a)],
        out_specs=[pl.BlockSpec((_BN, _H), lambda i: (i, 0)),
                   pl.BlockSpec((_BN, 2 * _H), lambda i: (i, 0))],
        out_shape=[jax.ShapeDtypeStruct((_N, _H), jnp.float32),
                   jax.ShapeDtypeStruct((_N, 2 * _H), jnp.float32)],
    )(nf, ag4, w1, b1, w2, b2, w3, b3, w1a)


def _pad_cols(a, to):
    return jnp.pad(a, ((0, 0), (0, to - a.shape[1])))


def _pad_rows(a, to):
    return jnp.pad(a, ((0, to - a.shape[0]), (0, 0)))


def _b2(b):
    return b.reshape(1, -1)


def kernel(x, edge_index, edge_attr, params):
    dst = edge_index[1].astype(jnp.int32)
    dst2d = dst.reshape(_NROWS, _ROW)
    dst2d_pad = jnp.pad(dst2d, ((0, _NROWS_PAD - _NROWS), (0, 0)))

    mp = params["mp"]
    w1a = [layer["lin_edge"][0][0][:_H] for layer in mp]

    (wa1, ba1), (wa2, ba2), (wa3, ba3) = params["node_in"]
    nf, p128 = _tc_node_in(_pad_cols(x, 16), _pad_rows(wa1, 16), _b2(ba1),
                           wa2, _b2(ba2), wa3, _b2(ba3), w1a[0])
    (we1, be1), (we2, be2), (we3, be3) = params["edge_in"]
    me = _mlp3(_pad_cols(edge_attr, 8), _pad_rows(we1, 8), _b2(be1),
               we2, _b2(be2), we3, _b2(be3), _BE, pad_left=True)

    for li, layer in enumerate(mp):
        (u1, c1), (u2, c2), (u3, c3) = layer["lin_edge"]
        (v1, d1), (v2, d2), (v3, d3) = layer["lin_node"]
        g = _sc_gather(p128, dst2d_pad)
        me = _tc_edge(g, me, u1[_H:], _b2(c1), u2, _b2(c2), u3, _b2(c3))
        ag4 = _sc_scatter_add(me, dst2d)
        w1a_next = w1a[li + 1] if li + 1 < len(mp) else w1a[0]
        nf, p128 = _tc_node(nf, ag4, v1, _b2(d1), v2, _b2(d2), v3, _b2(d3),
                            w1a_next)

    (wo1, bo1), (wo2, bo2), (wo3, bo3) = params["node_out"]
    out8 = _mlp3(nf, wo1, _b2(bo1), wo2, _b2(bo2),
                 _pad_cols(wo3, 8), _pad_cols(_b2(bo3), 8), _BN)
    return out8[:, :2]


# speedup vs baseline: 1.6749x; 1.0633x over previous
import functools

import jax
import jax.numpy as jnp
from jax import lax
from jax.experimental import pallas as pl
from jax.experimental.pallas import tpu as pltpu
from jax.experimental.pallas import tpu_sc as plsc

_N = 50000
_E = 800000
_H = 64
_ROW = 128
_NROWS = _E // _ROW
_NROWS_PAD = 6400
_ZCH = 125
_NCH = _N // _ZCH
_BE = 4000
_BN = 2000


def _mesh():
    return plsc.VectorSubcoreMesh(core_axis_name="core", subcore_axis_name="subcore")


def _sc_gather(table, idx2d):
    rows, width = idx2d.shape
    d = table.shape[1]

    perw = rows // 32

    @functools.partial(
        pl.kernel,
        out_type=jax.ShapeDtypeStruct((rows * width, d), table.dtype),
        mesh=_mesh(),
        scratch_types=[
            pltpu.VMEM((2, 1, _ROW), jnp.int32),
            pltpu.VMEM((2, _ROW, 128), jnp.float32),
            pltpu.SemaphoreType.DMA((2,)),
            pltpu.SemaphoreType.DMA((2,)),
            pltpu.SemaphoreType.DMA((2,)),
        ],
    )
    def k(tab_hbm, i_hbm, o_hbm, idx_v, g_v, sem_i, sem_g, sem_o):
        c = lax.axis_index("core")
        s = lax.axis_index("subcore")
        base = (s * 2 + c) * perw

        def icp(r, b):
            return pltpu.make_async_copy(i_hbm.at[pl.ds(base + r, 1), :],
                                         idx_v.at[b], sem_i.at[b])

        def gcp(r, b):
            return pltpu.make_async_copy(tab_hbm.at[idx_v.at[b, 0]],
                                         g_v.at[b], sem_g.at[b])

        def ocp(r, b):
            return pltpu.make_async_copy(
                g_v.at[b], o_hbm.at[pl.ds((base + r) * _ROW, _ROW), :],
                sem_o.at[b])

        icp(0, 0).start()
        icp(1, 1).start()

        @pl.loop(0, perw, step=2)
        def _grp(r):
            for b in range(2):
                @pl.when(r + b >= 2)
                def _w1():
                    ocp(r + b - 2, b).wait()
                icp(r + b, b).wait()
                gcp(r + b, b).start()
                gcp(r + b, b).wait()
                ocp(r + b, b).start()

                @pl.when(r + b + 2 < perw)
                def _w2():
                    icp(r + b + 2, b).start()

        ocp(perw - 2, 0).wait()
        ocp(perw - 1, 1).wait()

    return k(table, idx2d)


def _sc_scatter_add(msg128, dst2d):
    @functools.partial(
        pl.kernel,
        out_type=jax.ShapeDtypeStruct((4, _N, 128), jnp.float32),
        mesh=_mesh(),
        compiler_params=pltpu.CompilerParams(use_tc_tiling_on_sc=False),
        scratch_types=[
            pltpu.VMEM_SHARED((_N, 16), jnp.float32),
            pltpu.VMEM((8, 1, _ROW), jnp.int32),
            pltpu.VMEM((8, _ROW, 16), jnp.float32),
            pltpu.VMEM((_ZCH, 16), jnp.float32),
            pltpu.VMEM((_ZCH, 128), jnp.float32),
            pltpu.SemaphoreType.DMA((8,)),
            pltpu.SemaphoreType.DMA((8,)),
        ],
    )
    def k(m_hbm, i_hbm, o_hbm, acc, idx_v, m_v, z_v, stage_v, sem_in, sem_add):
        c = lax.axis_index("core")
        s = lax.axis_index("subcore")
        nk = (_NROWS + 15) // 16
        nk_pad = ((nk + 7) // 8) * 8

        @pl.loop(0, _ZCH)
        def _z(i):
            z_v[i, :] = jnp.zeros((16,), jnp.float32)

        def in_cps(k_i, b, q_s):
            r = s + 16 * k_i
            return (
                pltpu.make_async_copy(i_hbm.at[pl.ds(r, 1), :], idx_v.at[b],
                                      sem_in.at[b]),
                pltpu.make_async_copy(
                    m_hbm.at[pl.ds(r * _ROW, _ROW), pl.ds(16 * q_s, 16)],
                    m_v.at[b], sem_in.at[b]),
            )

        def add_cp(b):
            return pltpu.make_async_copy(m_v.at[b], acc.at[idx_v.at[b, 0]],
                                         sem_add.at[b])

        def start_in(k_i, b, q_s):
            @pl.when(s + 16 * k_i < _NROWS)
            def _():
                a, m = in_cps(k_i, b, q_s)
                a.start()
                m.start()

        def one_pass(q_s):
            @pl.loop(s, _NCH, step=16)
            def _zero(ch):
                pltpu.sync_copy(z_v, acc.at[pl.ds(ch * _ZCH, _ZCH), :])

            plsc.subcore_barrier()

            for j in range(4):
                start_in(j, j, q_s)

            @pl.loop(0, nk_pad, step=8)
            def _grp(k0):
                for u in range(8):
                    k_i = k0 + u
                    b = u
                    b4 = (u + 4) % 8
                    ok = s + 16 * k_i < _NROWS

                    @pl.when(ok)
                    def _go():
                        a, m = in_cps(k_i, b, q_s)
                        a.wait()
                        m.wait()
                        pltpu.async_copy(m_v.at[b], acc.at[idx_v.at[b, 0]],
                                         sem_add.at[b], add=True)

                    @pl.when(jnp.logical_and(k_i >= 4,
                                             s + 16 * (k_i - 4) < _NROWS))
                    def _free():
                        add_cp(b4).wait()

                    @pl.when(s + 16 * (k_i + 4) < _NROWS)
                    def _pref():
                        a, m = in_cps(k_i + 4, b4, q_s)
                        a.start()
                        m.start()

            for kk in range(nk_pad - 4, nk):
                @pl.when(s + 16 * kk < _NROWS)
                def _dr():
                    add_cp(kk % 8).wait()

            plsc.subcore_barrier()

            @pl.loop(s, _NCH, step=16)
            def _out(ch):
                pltpu.sync_copy(acc.at[pl.ds(ch * _ZCH, _ZCH), :],
                                stage_v.at[:, pl.ds(0, 16)])
                pltpu.sync_copy(stage_v, o_hbm.at[q_s, pl.ds(ch * _ZCH, _ZCH), :])

            plsc.subcore_barrier()

        for q_off in range(2):
            for c_s in range(2):
                @pl.when(c == c_s)
                def _p():
                    one_pass(2 * c_s + q_off)

    return k(msg128, dst2d)


def _dot(a, b):
    return jnp.dot(a, b, preferred_element_type=jnp.float32)


def _full(a):
    return pl.BlockSpec(a.shape, lambda i: tuple(0 for _ in a.shape))


def _mlp3(xp, w1, b1, w2, b2, w3, b3, blk, pad_left=False):
    r, k0 = xp.shape
    ko = w3.shape[1]

    def body(x_ref, w1r, b1r, w2r, b2r, w3r, b3r, o_ref):
        h = jnp.maximum(_dot(x_ref[...], w1r[...]) + b1r[...], 0.0)
        h = jnp.maximum(_dot(h, w2r[...]) + b2r[...], 0.0)
        o = _dot(h, w3r[...]) + b3r[...]
        if pad_left:
            o = jnp.concatenate([jnp.zeros_like(o), o], axis=1)
        o_ref[...] = o

    if pad_left:
        ko *= 2
    return pl.pallas_call(
        body,
        grid=(r // blk,),
        in_specs=[pl.BlockSpec((blk, k0), lambda i: (i, 0)),
                  _full(w1), _full(b1), _full(w2), _full(b2), _full(w3), _full(b3)],
        out_specs=pl.BlockSpec((blk, ko), lambda i: (i, 0)),
        out_shape=jax.ShapeDtypeStruct((r, ko), jnp.float32),
    )(xp, w1, b1, w2, b2, w3, b3)


def _tc_node_in(xp, w1, b1, w2, b2, w3, b3, w1a):
    r, k0 = xp.shape

    def body(x_ref, w1r, b1r, w2r, b2r, w3r, b3r, w1ar, nf_ref, p_ref):
        h = jnp.maximum(_dot(x_ref[...], w1r[...]) + b1r[...], 0.0)
        h = jnp.maximum(_dot(h, w2r[...]) + b2r[...], 0.0)
        nf = _dot(h, w3r[...]) + b3r[...]
        nf_ref[...] = nf
        p = _dot(nf, w1ar[...])
        p_ref[...] = jnp.concatenate([p, jnp.zeros_like(p)], axis=1)

    return pl.pallas_call(
        body,
        grid=(r // _BN,),
        in_specs=[pl.BlockSpec((_BN, k0), lambda i: (i, 0)),
                  _full(w1), _full(b1), _full(w2), _full(b2), _full(w3),
                  _full(b3), _full(w1a)],
        out_specs=[pl.BlockSpec((_BN, _H), lambda i: (i, 0)),
                   pl.BlockSpec((_BN, 2 * _H), lambda i: (i, 0))],
        out_shape=[jax.ShapeDtypeStruct((r, _H), jnp.float32),
                   jax.ShapeDtypeStruct((r, 2 * _H), jnp.float32)],
    )(xp, w1, b1, w2, b2, w3, b3, w1a)


def _tc_edge(g128, me, w1b, b1, w2, b2, w3, b3):
    def body(g_ref, me_ref, w1br, b1r, w2r, b2r, w3r, b3r, o_ref):
        p = g_ref[...][:, :_H]
        ef = me_ref[...][:, _H:]
        h = jnp.maximum(p + _dot(ef, w1br[...]) + b1r[...], 0.0)
        h = jnp.maximum(_dot(h, w2r[...]) + b2r[...], 0.0)
        m = _dot(h, w3r[...]) + b3r[...]
        o_ref[...] = jnp.concatenate([m, ef + m], axis=1)

    return pl.pallas_call(
        body,
        grid=(_E // _BE,),
        in_specs=[pl.BlockSpec((_BE, 2 * _H), lambda i: (i, 0)),
                  pl.BlockSpec((_BE, 2 * _H), lambda i: (i, 0)),
                  _full(w1b), _full(b1), _full(w2), _full(b2), _full(w3), _full(b3)],
        out_specs=pl.BlockSpec((_BE, 2 * _H), lambda i: (i, 0)),
        out_shape=jax.ShapeDtypeStruct((_E, 2 * _H), jnp.float32),
    )(g128, me, w1b, b1, w2, b2, w3, b3)


def _tc_node(nf, ag4, w1, b1, w2, b2, w3, b3, w1a):
    def body(nf_ref, ag_ref, w1r, b1r, w2r, b2r, w3r, b3r, w1ar, nfn_ref, p_ref):
        ag = ag_ref[...]
        cat = jnp.concatenate(
            [nf_ref[...], ag[0, :, :16], ag[1, :, :16], ag[2, :, :16],
             ag[3, :, :16]], axis=1)
        h = jnp.maximum(_dot(cat, w1r[...]) + b1r[...], 0.0)
        h = jnp.maximum(_dot(h, w2r[...]) + b2r[...], 0.0)
        nfn = nf_ref[...] + _dot(h, w3r[...]) + b3r[...]
        nfn_ref[...] = nfn
        p = _dot(nfn, w1ar[...])
        p_ref[...] = jnp.concatenate([p, jnp.zeros_like(p)], axis=1)

    return pl.pallas_call(
        body,
        grid=(_N // _BN,),
        in_specs=[pl.BlockSpec((_BN, _H), lambda i: (i, 0)),
                  pl.BlockSpec((4, _BN, 128), lambda i: (0, i, 0)),
                  _full(w1), _full(b1), _full(w2), _full(b2), _full(w3),
                  _full(b3), _full(w1a)],
        out_specs=[pl.BlockSpec((_BN, _H), lambda i: (i, 0)),
                   pl.BlockSpec((_BN, 2 * _H), lambda i: (i, 0))],
        out_shape=[jax.ShapeDtypeStruct((_N, _H), jnp.float32),
                   jax.ShapeDtypeStruct((_N, 2 * _H), jnp.float32)],
    )(nf, ag4, w1, b1, w2, b2, w3, b3, w1a)


def _pad_cols(a, to):
    return jnp.pad(a, ((0, 0), (0, to - a.shape[1])))


def _pad_rows(a, to):
    return jnp.pad(a, ((0, to - a.shape[0]), (0, 0)))


def _b2(b):
    return b.reshape(1, -1)


def kernel(x, edge_index, edge_attr, params):
    dst = edge_index[1].astype(jnp.int32)
    dst2d = dst.reshape(_NROWS, _ROW)
    dst2d_pad = jnp.pad(dst2d, ((0, _NROWS_PAD - _NROWS), (0, 0)))

    mp = params["mp"]
    w1a = [layer["lin_edge"][0][0][:_H] for layer in mp]

    (wa1, ba1), (wa2, ba2), (wa3, ba3) = params["node_in"]
    nf, p128 = _tc_node_in(_pad_cols(x, 16), _pad_rows(wa1, 16), _b2(ba1),
                           wa2, _b2(ba2), wa3, _b2(ba3), w1a[0])
    (we1, be1), (we2, be2), (we3, be3) = params["edge_in"]
    me = _mlp3(_pad_cols(edge_attr, 8), _pad_rows(we1, 8), _b2(be1),
               we2, _b2(be2), we3, _b2(be3), _BE, pad_left=True)

    for li, layer in enumerate(mp):
        (u1, c1), (u2, c2), (u3, c3) = layer["lin_edge"]
        (v1, d1), (v2, d2), (v3, d3) = layer["lin_node"]
        g = _sc_gather(p128, dst2d_pad)
        me = _tc_edge(g, me, u1[_H:], _b2(c1), u2, _b2(c2), u3, _b2(c3))
        ag4 = _sc_scatter_add(me, dst2d)
        w1a_next = w1a[li + 1] if li + 1 < len(mp) else w1a[0]
        nf, p128 = _tc_node(nf, ag4, v1, _b2(d1), v2, _b2(d2), v3, _b2(d3),
                            w1a_next)

    (wo1, bo1), (wo2, bo2), (wo3, bo3) = params["node_out"]
    out8 = _mlp3(nf, wo1, _b2(bo1), wo2, _b2(bo2),
                 _pad_cols(wo3, 8), _pad_cols(_b2(bo3), 8), _BN)
    return out8[:, :2]
